# Initial kernel scaffold; baseline (speedup 1.0000x reference)
import jax
import jax.numpy as jnp
from jax.experimental import pallas as pl


def kernel(x, params, edge_index):
    raise NotImplementedError("write your pallas kernel here")


# speedup vs baseline: 5.2493x; 5.2493x over previous
import functools

import jax
import jax.numpy as jnp
from jax import lax
from jax.experimental import pallas as pl
from jax.experimental.pallas import tpu as pltpu
from jax.experimental.pallas import tpu_sc as plsc

N = 10000
E = 320000
L = 128

NC = 2
NS = 16
NW = NC * NS
EDGES_PER_TILE = E // NW
CHUNK = 125
NCHUNK = EDGES_PER_TILE // CHUNK
NPAD = 10240
ROWS_PER_TILE = NPAD // NS


def _sc_agg_body(h_hbm, send_hbm, recv_hbm, zero_hbm, out_hbm,
                 send_v, recv_v, rows_v, agg_sh, gsem):
    cid = lax.axis_index("c")
    sid = lax.axis_index("s")
    wid = cid * NS + sid

    pltpu.sync_copy(send_hbm.at[wid], send_v)
    pltpu.sync_copy(recv_hbm.at[wid], recv_v)

    row0 = sid * ROWS_PER_TILE
    pltpu.sync_copy(zero_hbm, agg_sh.at[pl.ds(row0, ROWS_PER_TILE)])
    plsc.subcore_barrier()

    def body(j, carry):
        pltpu.async_copy(h_hbm.at[send_v.at[j]], rows_v, gsem).wait()
        pltpu.sync_copy(rows_v, agg_sh.at[recv_v.at[j]], add=True)
        return carry

    lax.fori_loop(0, NCHUNK, body, 0, unroll=False)
    plsc.subcore_barrier()

    pltpu.sync_copy(agg_sh.at[pl.ds(row0, ROWS_PER_TILE)],
                    out_hbm.at[cid, pl.ds(row0, ROWS_PER_TILE)])


@functools.lru_cache(maxsize=1)
def _make_sc_agg():
    return pl.kernel(
        _sc_agg_body,
        out_type=jax.ShapeDtypeStruct((NC, NPAD, L), jnp.float32),
        mesh=plsc.VectorSubcoreMesh(core_axis_name="c", subcore_axis_name="s"),
        scratch_types=[
            pltpu.VMEM((NCHUNK, CHUNK), jnp.int32),
            pltpu.VMEM((NCHUNK, CHUNK), jnp.int32),
            pltpu.VMEM((CHUNK, L), jnp.float32),
            pltpu.VMEM_SHARED((NPAD, L), jnp.float32),
            pltpu.SemaphoreType.DMA,
        ],
    )


def _sc_agg(h, send, recv, zero):
    return _make_sc_agg()(h, send, recv, zero)


BN = 1000
GN = N // BN


def _ln(u, g, b):
    m = jnp.mean(u, axis=-1, keepdims=True)
    v = jnp.mean((u - m) * (u - m), axis=-1, keepdims=True)
    return (u - m) * lax.rsqrt(v + 1e-5) * g + b


def _dot(a, w):
    return jnp.dot(a, w, preferred_element_type=jnp.float32,
                   precision=lax.Precision.HIGHEST)


def _enc_tc_body(x_ref, w0, w1, w2, w3, b0, b1, b2, b3, g, bl, out_ref):
    u = x_ref[...]
    u = jax.nn.relu(_dot(u, w0[...]) + b0[...])
    u = jax.nn.relu(_dot(u, w1[...]) + b1[...])
    u = jax.nn.relu(_dot(u, w2[...]) + b2[...])
    u = _dot(u, w3[...]) + b3[...]
    out_ref[...] = _ln(u, g[...], bl[...])


def _mp_tc_body(h_ref, p_ref, w0a, w0b, w1, w2, w3,
                b0, b1, b2, b3, g, bl, out_ref):
    h = h_ref[...]
    agg = p_ref[0] + p_ref[1]
    u = jax.nn.relu(_dot(h, w0a[...]) + _dot(agg, w0b[...]) + b0[...])
    u = jax.nn.relu(_dot(u, w1[...]) + b1[...])
    u = jax.nn.relu(_dot(u, w2[...]) + b2[...])
    u = _dot(u, w3[...]) + b3[...]
    out_ref[...] = _ln(u, g[...], bl[...])


def _dec_tc_body(h_ref, w0, w1, w2, w3, b0, b1, b2, b3, out_ref):
    u = h_ref[...]
    u = jax.nn.relu(_dot(u, w0[...]) + b0[...])
    u = jax.nn.relu(_dot(u, w1[...]) + b1[...])
    u = jax.nn.relu(_dot(u, w2[...]) + b2[...])
    out_ref[...] = _dot(u, w3[...]) + b3[...]


def _row_spec(bn, d):
    return pl.BlockSpec((bn, d), lambda i: (i, 0))


def _full_spec(shape):
    nd = len(shape)
    return pl.BlockSpec(shape, lambda i: (0,) * nd)


def _enc_tc(x, Ws, bs, g, b):
    args = [x] + list(Ws) + [bb.reshape(1, L) for bb in bs] + [
        g.reshape(1, L), b.reshape(1, L)]
    specs = [_row_spec(BN, L)] + [_full_spec(w.shape) for w in args[1:]]
    return pl.pallas_call(
        _enc_tc_body,
        grid=(GN,),
        in_specs=specs,
        out_specs=_row_spec(BN, L),
        out_shape=jax.ShapeDtypeStruct((N, L), jnp.float32),
    )(*args)


def _mp_tc(h, p, Ws, bs, g, b):
    w0a = Ws[0][:L]
    w0b = Ws[0][L:]
    args = [h, p, w0a, w0b, Ws[1], Ws[2], Ws[3]] + [
        bb.reshape(1, L) for bb in bs] + [g.reshape(1, L), b.reshape(1, L)]
    specs = ([_row_spec(BN, L),
              pl.BlockSpec((NC, BN, L), lambda i: (0, i, 0))] +
             [_full_spec(a.shape) for a in args[2:]])
    return pl.pallas_call(
        _mp_tc_body,
        grid=(GN,),
        in_specs=specs,
        out_specs=_row_spec(BN, L),
        out_shape=jax.ShapeDtypeStruct((N, L), jnp.float32),
    )(h, p, *args[2:])


def _dec_tc(h, Ws, bs):
    args = [h] + list(Ws) + [bb.reshape(1, L) for bb in bs]
    specs = [_row_spec(BN, L)] + [_full_spec(a.shape) for a in args[1:]]
    return pl.pallas_call(
        _dec_tc_body,
        grid=(GN,),
        in_specs=specs,
        out_specs=_row_spec(BN, L),
        out_shape=jax.ShapeDtypeStruct((N, L), jnp.float32),
    )(*args)


def kernel(x, params, edge_index):
    send = edge_index[0].astype(jnp.int32).reshape(NW, NCHUNK, CHUNK)
    recv = edge_index[1].astype(jnp.int32).reshape(NW, NCHUNK, CHUNK)
    zero = jnp.zeros((ROWS_PER_TILE, L), jnp.float32)

    enc = params["enc"]
    h = _enc_tc(x, enc["Ws"], enc["bs"], enc["g"], enc["b"])
    for mp in params["mps"]:
        p = _sc_agg(h, send, recv, zero)
        h = _mp_tc(h, p, mp["Ws"], mp["bs"], mp["g"], mp["b"])
    dec = params["dec"]
    out = _dec_tc(h, dec["Ws"], dec["bs"])
    return out[:, 1:3]


# speedup vs baseline: 6.2745x; 1.1953x over previous
import functools

import jax
import jax.numpy as jnp
from jax import lax
from jax.experimental import pallas as pl
from jax.experimental.pallas import tpu as pltpu
from jax.experimental.pallas import tpu_sc as plsc

N = 10000
E = 320000
L = 128

NC = 2
NS = 16
NW = NC * NS
EDGES_PER_TILE = E // NW
CHUNK = 125
NCHUNK = EDGES_PER_TILE // CHUNK
NPAD = 10240
ROWS_PER_TILE = NPAD // NS


def _sc_agg_body(h_hbm, send_hbm, recv_hbm, zero_hbm, out_hbm,
                 sendi, recvi, rows_v, agg_sh, gsem, ssem, isem):
    cid = lax.axis_index("c")
    sid = lax.axis_index("s")
    wid = cid * NS + sid

    row0 = sid * ROWS_PER_TILE
    pltpu.sync_copy(zero_hbm, agg_sh.at[pl.ds(row0, ROWS_PER_TILE)])

    NG = NCHUNK // 2

    def _ipull(g, slot):
        pltpu.async_copy(send_hbm.at[wid, pl.ds(2 * g, 2)], sendi.at[slot], isem)
        pltpu.async_copy(recv_hbm.at[wid, pl.ds(2 * g, 2)], recvi.at[slot], isem)

    def _iwait(g, slot):
        pltpu.make_async_copy(send_hbm.at[wid, pl.ds(2 * g, 2)], sendi.at[slot],
                              isem).wait()
        pltpu.make_async_copy(recv_hbm.at[wid, pl.ds(2 * g, 2)], recvi.at[slot],
                              isem).wait()

    def _gather(slot, b, buf):
        return pltpu.async_copy(h_hbm.at[sendi.at[slot, b]], rows_v.at[buf],
                                gsem)

    def _gwait(slot, b, buf):
        pltpu.make_async_copy(h_hbm.at[sendi.at[slot, b]], rows_v.at[buf],
                              gsem).wait()

    def _scatter(slot, b, buf):
        return pltpu.async_copy(rows_v.at[buf], agg_sh.at[recvi.at[slot, b]],
                                ssem, add=True)

    def _swait(slot, b, buf):
        pltpu.make_async_copy(rows_v.at[buf], agg_sh.at[recvi.at[slot, b]],
                              ssem).wait()

    pltpu.sync_copy(send_hbm.at[wid, pl.ds(0, 2)], sendi.at[0])
    pltpu.sync_copy(recv_hbm.at[wid, pl.ds(0, 2)], recvi.at[0])
    plsc.subcore_barrier()
    _gather(0, 0, 0)

    def group(g, carry):
        gb = lax.rem(g, 2)
        nb = 1 - gb
        _gwait(gb, 0, 0)

        @pl.when(g > 0)
        def _():
            _swait(nb, 1, 1)

        @pl.when(g < NG - 1)
        def _():
            _ipull(g + 1, nb)

        _scatter(gb, 0, 0)
        _gather(gb, 1, 1)
        _gwait(gb, 1, 1)
        _swait(gb, 0, 0)
        _scatter(gb, 1, 1)

        @pl.when(g < NG - 1)
        def _():
            _iwait(g + 1, nb)
            _gather(nb, 0, 0)

        return carry

    lax.fori_loop(0, NG, group, 0, unroll=False)
    _swait(lax.rem(NG - 1, 2), 1, 1)
    plsc.subcore_barrier()

    pltpu.sync_copy(agg_sh.at[pl.ds(row0, ROWS_PER_TILE)],
                    out_hbm.at[cid, pl.ds(row0, ROWS_PER_TILE)])


@functools.lru_cache(maxsize=1)
def _make_sc_agg():
    return pl.kernel(
        _sc_agg_body,
        out_type=jax.ShapeDtypeStruct((NC, NPAD, L), jnp.float32),
        mesh=plsc.VectorSubcoreMesh(core_axis_name="c", subcore_axis_name="s"),
        scratch_types=[
            pltpu.VMEM((2, 2, CHUNK), jnp.int32),
            pltpu.VMEM((2, 2, CHUNK), jnp.int32),
            pltpu.VMEM((2, CHUNK, L), jnp.float32),
            pltpu.VMEM_SHARED((NPAD, L), jnp.float32),
            pltpu.SemaphoreType.DMA,
            pltpu.SemaphoreType.DMA,
            pltpu.SemaphoreType.DMA,
        ],
    )


def _sc_agg(h, send, recv, zero):
    return _make_sc_agg()(h, send, recv, zero)


BN = 1000
GN = N // BN


def _ln(u, g, b):
    m = jnp.mean(u, axis=-1, keepdims=True)
    v = jnp.mean((u - m) * (u - m), axis=-1, keepdims=True)
    return (u - m) * lax.rsqrt(v + 1e-5) * g + b


def _dot(a, w):
    return jnp.dot(a, w, preferred_element_type=jnp.float32,
                   precision=lax.Precision.HIGHEST)


def _enc_tc_body(x_ref, w0, w1, w2, w3, b0, b1, b2, b3, g, bl, out_ref):
    u = x_ref[...]
    u = jax.nn.relu(_dot(u, w0[...]) + b0[...])
    u = jax.nn.relu(_dot(u, w1[...]) + b1[...])
    u = jax.nn.relu(_dot(u, w2[...]) + b2[...])
    u = _dot(u, w3[...]) + b3[...]
    out_ref[...] = _ln(u, g[...], bl[...])


def _mp_tc_body(h_ref, p_ref, w0a, w0b, w1, w2, w3,
                b0, b1, b2, b3, g, bl, out_ref):
    h = h_ref[...]
    agg = p_ref[0] + p_ref[1]
    u = jax.nn.relu(_dot(h, w0a[...]) + _dot(agg, w0b[...]) + b0[...])
    u = jax.nn.relu(_dot(u, w1[...]) + b1[...])
    u = jax.nn.relu(_dot(u, w2[...]) + b2[...])
    u = _dot(u, w3[...]) + b3[...]
    out_ref[...] = _ln(u, g[...], bl[...])


def _dec_tc_body(h_ref, w0, w1, w2, w3, b0, b1, b2, b3, out_ref):
    u = h_ref[...]
    u = jax.nn.relu(_dot(u, w0[...]) + b0[...])
    u = jax.nn.relu(_dot(u, w1[...]) + b1[...])
    u = jax.nn.relu(_dot(u, w2[...]) + b2[...])
    out_ref[...] = _dot(u, w3[...]) + b3[...]


def _row_spec(bn, d):
    return pl.BlockSpec((bn, d), lambda i: (i, 0))


def _full_spec(shape):
    nd = len(shape)
    return pl.BlockSpec(shape, lambda i: (0,) * nd)


def _enc_tc(x, Ws, bs, g, b):
    args = [x] + list(Ws) + [bb.reshape(1, L) for bb in bs] + [
        g.reshape(1, L), b.reshape(1, L)]
    specs = [_row_spec(BN, L)] + [_full_spec(w.shape) for w in args[1:]]
    return pl.pallas_call(
        _enc_tc_body,
        grid=(GN,),
        in_specs=specs,
        out_specs=_row_spec(BN, L),
        out_shape=jax.ShapeDtypeStruct((N, L), jnp.float32),
    )(*args)


def _mp_tc(h, p, Ws, bs, g, b):
    w0a = Ws[0][:L]
    w0b = Ws[0][L:]
    args = [h, p, w0a, w0b, Ws[1], Ws[2], Ws[3]] + [
        bb.reshape(1, L) for bb in bs] + [g.reshape(1, L), b.reshape(1, L)]
    specs = ([_row_spec(BN, L),
              pl.BlockSpec((NC, BN, L), lambda i: (0, i, 0))] +
             [_full_spec(a.shape) for a in args[2:]])
    return pl.pallas_call(
        _mp_tc_body,
        grid=(GN,),
        in_specs=specs,
        out_specs=_row_spec(BN, L),
        out_shape=jax.ShapeDtypeStruct((N, L), jnp.float32),
    )(h, p, *args[2:])


def _dec_tc(h, Ws, bs):
    args = [h] + list(Ws) + [bb.reshape(1, L) for bb in bs]
    specs = [_row_spec(BN, L)] + [_full_spec(a.shape) for a in args[1:]]
    return pl.pallas_call(
        _dec_tc_body,
        grid=(GN,),
        in_specs=specs,
        out_specs=_row_spec(BN, L),
        out_shape=jax.ShapeDtypeStruct((N, L), jnp.float32),
    )(*args)


def kernel(x, params, edge_index):
    send = edge_index[0].astype(jnp.int32).reshape(NW, NCHUNK, CHUNK)
    recv = edge_index[1].astype(jnp.int32).reshape(NW, NCHUNK, CHUNK)
    zero = jnp.zeros((ROWS_PER_TILE, L), jnp.float32)

    enc = params["enc"]
    h = _enc_tc(x, enc["Ws"], enc["bs"], enc["g"], enc["b"])
    for mp in params["mps"]:
        p = _sc_agg(h, send, recv, zero)
        h = _mp_tc(h, p, mp["Ws"], mp["bs"], mp["g"], mp["b"])
    dec = params["dec"]
    out = _dec_tc(h, dec["Ws"], dec["bs"])
    return out[:, 1:3]


# speedup vs baseline: 8.4260x; 1.3429x over previous
import functools

import jax
import jax.numpy as jnp
from jax import lax
from jax.experimental import pallas as pl
from jax.experimental.pallas import tpu as pltpu
from jax.experimental.pallas import tpu_sc as plsc

N = 10000
E = 320000
L = 128

NC = 2
NS = 16
NW = NC * NS
EDGES_PER_TILE = E // NW
CHUNK = 125
NCHUNK = EDGES_PER_TILE // CHUNK
NPAD = 10240
ROWS_PER_TILE = NPAD // NS


def _sc_agg_body(h_hbm, send_hbm, recv_hbm, zero_hbm, out_hbm,
                 sendi, recvi, rows_v, agg_sh, gsem, ssem, isem):
    cid = lax.axis_index("c")
    sid = lax.axis_index("s")
    wid = cid * NS + sid

    row0 = sid * ROWS_PER_TILE
    pltpu.sync_copy(zero_hbm, agg_sh.at[pl.ds(row0, ROWS_PER_TILE)])

    NG = NCHUNK // 2

    def _ipull(g, slot):
        pltpu.async_copy(send_hbm.at[wid, pl.ds(2 * g, 2)], sendi.at[slot], isem)
        pltpu.async_copy(recv_hbm.at[wid, pl.ds(2 * g, 2)], recvi.at[slot], isem)

    def _iwait(g, slot):
        pltpu.make_async_copy(send_hbm.at[wid, pl.ds(2 * g, 2)], sendi.at[slot],
                              isem).wait()
        pltpu.make_async_copy(recv_hbm.at[wid, pl.ds(2 * g, 2)], recvi.at[slot],
                              isem).wait()

    def _gather(slot, b, buf):
        return pltpu.async_copy(h_hbm.at[sendi.at[slot, b]], rows_v.at[buf],
                                gsem)

    def _gwait(slot, b, buf):
        pltpu.make_async_copy(h_hbm.at[sendi.at[slot, b]], rows_v.at[buf],
                              gsem).wait()

    def _scatter(slot, b, buf):
        return pltpu.async_copy(rows_v.at[buf], agg_sh.at[recvi.at[slot, b]],
                                ssem, add=True)

    def _swait(slot, b, buf):
        pltpu.make_async_copy(rows_v.at[buf], agg_sh.at[recvi.at[slot, b]],
                              ssem).wait()

    pltpu.sync_copy(send_hbm.at[wid, pl.ds(0, 2)], sendi.at[0])
    pltpu.sync_copy(recv_hbm.at[wid, pl.ds(0, 2)], recvi.at[0])
    plsc.subcore_barrier()
    _gather(0, 0, 0)

    def group(g, carry):
        gb = lax.rem(g, 2)
        nb = 1 - gb
        _gwait(gb, 0, 0)

        @pl.when(g > 0)
        def _():
            _swait(nb, 1, 1)

        @pl.when(g < NG - 1)
        def _():
            _ipull(g + 1, nb)

        _scatter(gb, 0, 0)
        _gather(gb, 1, 1)
        _gwait(gb, 1, 1)
        _swait(gb, 0, 0)
        _scatter(gb, 1, 1)

        @pl.when(g < NG - 1)
        def _():
            _iwait(g + 1, nb)
            _gather(nb, 0, 0)

        return carry

    lax.fori_loop(0, NG, group, 0, unroll=False)
    _swait(lax.rem(NG - 1, 2), 1, 1)
    plsc.subcore_barrier()

    pltpu.sync_copy(agg_sh.at[pl.ds(row0, ROWS_PER_TILE)],
                    out_hbm.at[cid, pl.ds(row0, ROWS_PER_TILE)])


@functools.lru_cache(maxsize=1)
def _make_sc_agg():
    return pl.kernel(
        _sc_agg_body,
        out_type=jax.ShapeDtypeStruct((NC, NPAD, L), jnp.float32),
        mesh=plsc.VectorSubcoreMesh(core_axis_name="c", subcore_axis_name="s"),
        scratch_types=[
            pltpu.VMEM((2, 2, CHUNK), jnp.int32),
            pltpu.VMEM((2, 2, CHUNK), jnp.int32),
            pltpu.VMEM((2, CHUNK, L), jnp.float32),
            pltpu.VMEM_SHARED((NPAD, L), jnp.float32),
            pltpu.SemaphoreType.DMA,
            pltpu.SemaphoreType.DMA,
            pltpu.SemaphoreType.DMA,
        ],
    )


def _sc_agg(h, send, recv, zero):
    return _make_sc_agg()(h, send, recv, zero)


BN = 1000
GN = N // BN


def _ln(u, g, b):
    m = jnp.mean(u, axis=-1, keepdims=True)
    v = jnp.mean((u - m) * (u - m), axis=-1, keepdims=True)
    return (u - m) * lax.rsqrt(v + 1e-5) * g + b


def _split_w(w):
    hi = w.astype(jnp.bfloat16)
    lo = (w - hi.astype(jnp.float32)).astype(jnp.bfloat16)
    return jnp.stack([hi, lo])


def _dot(a, wp):
    a_hi = a.astype(jnp.bfloat16)
    a_lo = (a - a_hi.astype(jnp.float32)).astype(jnp.bfloat16)
    return (jnp.dot(a_hi, wp[0], preferred_element_type=jnp.float32)
            + jnp.dot(a_lo, wp[0], preferred_element_type=jnp.float32)
            + jnp.dot(a_hi, wp[1], preferred_element_type=jnp.float32))


def _enc_tc_body(x_ref, w0, w1, w2, w3, b0, b1, b2, b3, g, bl, out_ref):
    u = x_ref[...]
    u = jax.nn.relu(_dot(u, w0[...]) + b0[...])
    u = jax.nn.relu(_dot(u, w1[...]) + b1[...])
    u = jax.nn.relu(_dot(u, w2[...]) + b2[...])
    u = _dot(u, w3[...]) + b3[...]
    out_ref[...] = _ln(u, g[...], bl[...])


def _mp_tc_body(h_ref, p_ref, w0a, w0b, w1, w2, w3,
                b0, b1, b2, b3, g, bl, out_ref):
    h = h_ref[...]
    agg = p_ref[0] + p_ref[1]
    u = jax.nn.relu(_dot(h, w0a[...]) + _dot(agg, w0b[...]) + b0[...])
    u = jax.nn.relu(_dot(u, w1[...]) + b1[...])
    u = jax.nn.relu(_dot(u, w2[...]) + b2[...])
    u = _dot(u, w3[...]) + b3[...]
    out_ref[...] = _ln(u, g[...], bl[...])


def _dec_tc_body(h_ref, w0, w1, w2, w3, b0, b1, b2, b3, out_ref):
    u = h_ref[...]
    u = jax.nn.relu(_dot(u, w0[...]) + b0[...])
    u = jax.nn.relu(_dot(u, w1[...]) + b1[...])
    u = jax.nn.relu(_dot(u, w2[...]) + b2[...])
    out_ref[...] = _dot(u, w3[...]) + b3[...]


def _row_spec(bn, d):
    return pl.BlockSpec((bn, d), lambda i: (i, 0))


def _full_spec(shape):
    nd = len(shape)
    return pl.BlockSpec(shape, lambda i: (0,) * nd)


def _enc_tc(x, Ws, bs, g, b):
    args = [x] + [_split_w(w) for w in Ws] + [bb.reshape(1, L) for bb in bs] + [
        g.reshape(1, L), b.reshape(1, L)]
    specs = [_row_spec(BN, L)] + [_full_spec(w.shape) for w in args[1:]]
    return pl.pallas_call(
        _enc_tc_body,
        grid=(GN,),
        in_specs=specs,
        out_specs=_row_spec(BN, L),
        out_shape=jax.ShapeDtypeStruct((N, L), jnp.float32),
    )(*args)


def _mp_tc(h, p, Ws, bs, g, b):
    w0a = _split_w(Ws[0][:L])
    w0b = _split_w(Ws[0][L:])
    args = [h, p, w0a, w0b, _split_w(Ws[1]), _split_w(Ws[2]), _split_w(Ws[3])] + [
        bb.reshape(1, L) for bb in bs] + [g.reshape(1, L), b.reshape(1, L)]
    specs = ([_row_spec(BN, L),
              pl.BlockSpec((NC, BN, L), lambda i: (0, i, 0))] +
             [_full_spec(a.shape) for a in args[2:]])
    return pl.pallas_call(
        _mp_tc_body,
        grid=(GN,),
        in_specs=specs,
        out_specs=_row_spec(BN, L),
        out_shape=jax.ShapeDtypeStruct((N, L), jnp.float32),
    )(h, p, *args[2:])


def _dec_tc(h, Ws, bs):
    args = [h] + [_split_w(w) for w in Ws] + [bb.reshape(1, L) for bb in bs]
    specs = [_row_spec(BN, L)] + [_full_spec(a.shape) for a in args[1:]]
    return pl.pallas_call(
        _dec_tc_body,
        grid=(GN,),
        in_specs=specs,
        out_specs=_row_spec(BN, L),
        out_shape=jax.ShapeDtypeStruct((N, L), jnp.float32),
    )(*args)


def kernel(x, params, edge_index):
    send = edge_index[0].astype(jnp.int32).reshape(NW, NCHUNK, CHUNK)
    recv = edge_index[1].astype(jnp.int32).reshape(NW, NCHUNK, CHUNK)
    zero = jnp.zeros((ROWS_PER_TILE, L), jnp.float32)

    enc = params["enc"]
    h = _enc_tc(x, enc["Ws"], enc["bs"], enc["g"], enc["b"])
    for mp in params["mps"]:
        p = _sc_agg(h, send, recv, zero)
        h = _mp_tc(h, p, mp["Ws"], mp["bs"], mp["g"], mp["b"])
    dec = params["dec"]
    out = _dec_tc(h, dec["Ws"], dec["bs"])
    return out[:, 1:3]
